# baseline (speedup 1.0000x reference)
import functools

import jax
import jax.numpy as jnp
from jax import lax
from jax.experimental import pallas as pl
from jax.experimental.pallas import tpu as pltpu
from jax.experimental.pallas import tpu_sc as plsc

_NUM_CORES = 2
_NUM_SUBCORES = 16
_NUM_WORKERS = _NUM_CORES * _NUM_SUBCORES


@functools.partial(jax.jit, static_argnames=())
def _embed(x3, token_table, pos_table):
    B, nchunk, half = x3.shape
    L = nchunk * half
    V, D = token_table.shape
    rows_per_w = B // _NUM_WORKERS

    mesh = plsc.VectorSubcoreMesh(core_axis_name="c", subcore_axis_name="s")

    @functools.partial(
        pl.kernel,
        mesh=mesh,
        compiler_params=pltpu.CompilerParams(use_tc_tiling_on_sc=False),
        out_type=jax.ShapeDtypeStruct((B, L, D), jnp.float32),
        scratch_types=[
            pltpu.VMEM((nchunk, half), jnp.int32),
            pltpu.VMEM((L, D), jnp.float32),
            pltpu.SemaphoreType.DMA,
        ],
    )
    def k(x_hbm, tok_hbm, pos_hbm, out_hbm, idx_v, rows_v, sem):
        wid = lax.axis_index("s") * _NUM_CORES + lax.axis_index("c")
        base = wid * rows_per_w

        def body(i, carry):
            r = base + i
            pltpu.sync_copy(x_hbm.at[r], idx_v)
            pltpu.sync_copy(pos_hbm, rows_v)
            pltpu.async_copy(
                tok_hbm.at[idx_v.at[0]], rows_v.at[pl.ds(0, half)], sem, add=True
            ).wait()
            pltpu.async_copy(
                tok_hbm.at[idx_v.at[1]], rows_v.at[pl.ds(half, half)], sem, add=True
            ).wait()
            pltpu.sync_copy(rows_v, out_hbm.at[r])
            return carry

        lax.fori_loop(0, rows_per_w, body, 0)

    return k(x3, token_table, pos_table)


def kernel(x, token_table, pos_table):
    B, L = x.shape
    half = L // 2
    x3 = x.astype(jnp.int32).reshape(B, 2, half)
    return _embed(x3, token_table, pos_table)


# speedup vs baseline: 1.3763x; 1.3763x over previous
import functools

import jax
import jax.numpy as jnp
from jax import lax
from jax.experimental import pallas as pl
from jax.experimental.pallas import tpu as pltpu
from jax.experimental.pallas import tpu_sc as plsc

_NUM_CORES = 2
_NUM_SUBCORES = 16
_NUM_WORKERS = _NUM_CORES * _NUM_SUBCORES
_NROW = 4
_NIDX = 2


@jax.jit
def _embed(x3, token_table, pos_table):
    B, nchunk, half = x3.shape
    L = nchunk * half
    V, D = token_table.shape
    nrows = B // _NUM_WORKERS
    nout = nrows // _NROW

    mesh = plsc.VectorSubcoreMesh(core_axis_name="c", subcore_axis_name="s")

    @functools.partial(
        pl.kernel,
        mesh=mesh,
        compiler_params=pltpu.CompilerParams(use_tc_tiling_on_sc=False),
        out_type=jax.ShapeDtypeStruct((B, L, D), jnp.float32),
        scratch_types=(
            [pltpu.VMEM((nchunk, half), jnp.int32) for _ in range(_NIDX)]
            + [pltpu.VMEM((L, D), jnp.float32) for _ in range(_NROW)]
            + [
                pltpu.VMEM_SHARED((L, D), jnp.float32),
                pltpu.SemaphoreType.DMA((14,)),
            ]
        ),
    )
    def k(x_hbm, tok_hbm, pos_hbm, out_hbm,
          idx0, idx1, rows0, rows1, rows2, rows3, pos_sh, sem):
        idx_v = [idx0, idx1]
        rows_v = [rows0, rows1, rows2, rows3]
        wid = lax.axis_index("s") * _NUM_CORES + lax.axis_index("c")
        base = wid * nrows

        def idx_copy(s, b):
            return pltpu.make_async_copy(
                x_hbm.at[base + s], idx_v[b % _NIDX], sem.at[b % _NIDX])

        def pre_copy(s, b):
            del s
            return pltpu.make_async_copy(
                pos_sh, rows_v[b % _NROW], sem.at[2 + b % _NROW])

        def gather_copies(b):
            return [
                pltpu.make_async_copy(
                    tok_hbm.at[idx_v[b % _NIDX].at[c]],
                    rows_v[b % _NROW].at[pl.ds(c * half, half)],
                    sem.at[6 + b % _NROW],
                )
                for c in range(nchunk)
            ]

        def issue_gathers(b):
            for c in range(nchunk):
                pltpu.async_copy(
                    tok_hbm.at[idx_v[b % _NIDX].at[c]],
                    rows_v[b % _NROW].at[pl.ds(c * half, half)],
                    sem.at[6 + b % _NROW],
                    add=True,
                )

        def wait_gathers(b):
            for d in gather_copies(b):
                d.wait()

        def out_copy(s, b):
            return pltpu.make_async_copy(
                rows_v[b % _NROW], out_hbm.at[base + s], sem.at[10 + b % _NROW])

        def step(s, b, first, last):
            idx_copy(s, b).wait()
            pre_copy(s, b).wait()
            issue_gathers(b)
            if not (first and b == 0):
                wait_gathers(b - 1)
                out_copy(s - 1, b - 1).start()
            if not (last and b == _NROW - 1):
                idx_copy(s + 1, b + 1).start()
            if not (first and b in (0, 1)):
                out_copy(s - 2, b + 2).wait()
            if not (last and b in (_NROW - 2, _NROW - 1)):
                pre_copy(s + 2, b + 2).start()

        @pl.when(lax.axis_index("s") == 0)
        def _init():
            pltpu.sync_copy(pos_hbm, rows0)
            pltpu.sync_copy(rows0, pos_sh)

        plsc.subcore_barrier()

        idx_copy(0, 0).start()
        pre_copy(0, 0).start()
        pre_copy(1, 1).start()

        for b in range(_NROW):
            step(b, b, True, False)

        def body(gi, carry):
            for b in range(_NROW):
                step(gi * _NROW + b, b, False, False)
            return carry

        lax.fori_loop(1, nout - 1, body, 0)

        for b in range(_NROW):
            step((nout - 1) * _NROW + b, b, False, True)

        wait_gathers(_NROW - 1)
        out_copy(nrows - 1, _NROW - 1).start()
        out_copy(nrows - 2, _NROW - 2).wait()
        out_copy(nrows - 1, _NROW - 1).wait()

    return k(x3, token_table, pos_table)


def kernel(x, token_table, pos_table):
    B, L = x.shape
    half = L // 2
    x3 = x.astype(jnp.int32).reshape(B, 2, half)
    return _embed(x3, token_table, pos_table)
